# baseline (speedup 1.0000x reference)
import functools

import jax
import jax.numpy as jnp
from jax import lax
from jax.experimental import pallas as pl
from jax.experimental.pallas import tpu as pltpu
from jax.experimental.pallas import tpu_sc as plsc

N = 10000
E = 320000
D = 128
H = 256
C = 2

NC = 2
NS = 16
NW = NC * NS
LANES = 16

EPW = E // NW
CH = 80
NCH = EPW // CH
NP = 10240
RPT = NP // NS
ZBR = 128


def _sc_agg1_body(x_hbm, zrows_hbm, src_hbm, dst_hbm, msum_out, cnt_out,
                  src_v, dst_v, buf0, cnt_v, zrow_v, acc_s):
    cid = lax.axis_index("c")
    sid = lax.axis_index("s")

    zeros16 = jnp.zeros((LANES,), jnp.float32)
    ones16 = jnp.ones((LANES,), jnp.float32)

    pltpu.sync_copy(zrows_hbm, zrow_v)

    def _zcnt(i, c):
        cnt_v[pl.ds(i * LANES, LANES)] = zeros16
        return c
    lax.fori_loop(0, NP // LANES, _zcnt, 0)

    def _zacc(k, c):
        pltpu.sync_copy(zrow_v, acc_s.at[pl.ds(sid * RPT + k * ZBR, ZBR)])
        return c
    lax.fori_loop(0, RPT // ZBR, _zacc, 0)

    plsc.subcore_barrier()

    pltpu.sync_copy(src_hbm.at[cid].at[sid], src_v)
    pltpu.sync_copy(dst_hbm.at[cid].at[sid], dst_v)

    def _step(j, c):
        pltpu.sync_copy(x_hbm.at[src_v.at[j]], buf0)
        pltpu.sync_copy(buf0, acc_s.at[dst_v.at[j]], add=True)

        def _cstep(k, c2):
            idx = dst_v[j, pl.ds(k * LANES, LANES)]
            plsc.addupdate_scatter(cnt_v, [idx], ones16)
            return c2
        lax.fori_loop(0, CH // LANES, _cstep, 0)
        return c
    lax.fori_loop(0, NCH, _step, 0)

    plsc.subcore_barrier()

    pltpu.sync_copy(acc_s.at[pl.ds(sid * RPT, RPT)],
                    msum_out.at[cid].at[pl.ds(sid * RPT, RPT)])
    pltpu.sync_copy(cnt_v, cnt_out.at[pl.ds((cid * NS + sid) * NP, NP)])


def _sc_agg2_body(p0_hbm, p1_hbm, src_hbm, dst_hbm, out,
                  src_v, dst_v, p0_v, p1_v, a0_v, a1_v):
    cid = lax.axis_index("c")
    sid = lax.axis_index("s")
    wid = sid * NC + cid

    zeros16 = jnp.zeros((LANES,), jnp.float32)

    def _zacc(i, c):
        a0_v[pl.ds(i * LANES, LANES)] = zeros16
        a1_v[pl.ds(i * LANES, LANES)] = zeros16
        return c
    lax.fori_loop(0, N // LANES, _zacc, 0)

    pltpu.sync_copy(src_hbm.at[pl.ds(wid * EPW, EPW)], src_v)
    pltpu.sync_copy(dst_hbm.at[pl.ds(wid * EPW, EPW)], dst_v)
    pltpu.sync_copy(p0_hbm, p0_v)
    pltpu.sync_copy(p1_hbm, p1_v)

    def _step(t, c):
        s_idx = src_v[pl.ds(t * LANES, LANES)]
        d_idx = dst_v[pl.ds(t * LANES, LANES)]
        v0 = plsc.load_gather(p0_v, [s_idx])
        v1 = plsc.load_gather(p1_v, [s_idx])
        plsc.addupdate_scatter(a0_v, [d_idx], v0)
        plsc.addupdate_scatter(a1_v, [d_idx], v1)
        return c
    lax.fori_loop(0, EPW // LANES, _step, 0)

    pltpu.sync_copy(a0_v, out.at[pl.ds(wid * N, N)])
    pltpu.sync_copy(a1_v, out.at[pl.ds((NW + wid) * N, N)])


def _tc1_body(x_ref, ms_ref, cnt_ref, w1l_ref, b1_ref, w1r_ref, w2l_ref,
              w2r_ref, pt_ref):
    cnt = jnp.sum(cnt_ref[...], axis=0)[:N]
    inv = jnp.where(cnt > 0.0, 1.0 / jnp.maximum(cnt, 1.0), 0.0)
    ms = (ms_ref[0][:N].astype(jnp.float32) +
          ms_ref[1][:N].astype(jnp.float32))
    agg = ms * inv[:, None]
    h = lax.dot_general(agg, w1l_ref[...], (((1,), (1,)), ((), ())),
                        preferred_element_type=jnp.float32)
    h = h + b1_ref[...]
    h = h + lax.dot_general(x_ref[...].astype(jnp.float32), w1r_ref[...],
                            (((1,), (1,)), ((), ())),
                            preferred_element_type=jnp.float32)
    h = jnp.maximum(h, 0.0)
    p01 = lax.dot_general(w2l_ref[...], h, (((1,), (1,)), ((), ())),
                          preferred_element_type=jnp.float32)
    hr = lax.dot_general(w2r_ref[...], h, (((1,), (1,)), ((), ())),
                         preferred_element_type=jnp.float32)
    pt_ref[...] = jnp.concatenate(
        [p01, inv[None, :], hr, jnp.zeros((3, h.shape[0]), jnp.float32)],
        axis=0)


def _tc1(x, msum, cnt, W1_l, b1, W1_r, W2_l, W2_r):
    return pl.pallas_call(
        _tc1_body,
        out_shape=jax.ShapeDtypeStruct((8, N), jnp.float32),
    )(x, msum, cnt, W1_l, b1, W1_r, W2_l, W2_r)


def _tc2_body(m2_ref, pt_ref, b2_ref, out_ref):
    m2 = m2_ref[...]
    m0 = jnp.sum(m2[:NW], axis=0)
    m1 = jnp.sum(m2[NW:], axis=0)
    inv = pt_ref[2, :]
    z0 = m0 * inv + b2_ref[0, 0] + pt_ref[3, :]
    z1 = m1 * inv + b2_ref[0, 1] + pt_ref[4, :]
    mx = jnp.maximum(z0, z1)
    lse = mx + jnp.log(jnp.exp(z0 - mx) + jnp.exp(z1 - mx))
    out_ref[...] = jnp.concatenate([(z0 - lse)[None, :], (z1 - lse)[None, :]],
                                   axis=0)


def _tc2(m2, pt, b2):
    return pl.pallas_call(
        _tc2_body,
        out_shape=jax.ShapeDtypeStruct((C, N), jnp.float32),
    )(m2, pt, b2)


@functools.lru_cache(maxsize=1)
def _build_sc_kernels():
    mesh = plsc.VectorSubcoreMesh(
        core_axis_name="c", subcore_axis_name="s",
        num_cores=NC, num_subcores=NS)
    params = pltpu.CompilerParams(
        use_tc_tiling_on_sc=False, needs_layout_passes=False)
    sc_agg1 = pl.kernel(
        _sc_agg1_body,
        out_type=(
            jax.ShapeDtypeStruct((NC, NP, D), jnp.bfloat16),
            jax.ShapeDtypeStruct((NC * NS * NP,), jnp.float32),
        ),
        mesh=mesh,
        scratch_types=(
            pltpu.VMEM((NCH, CH), jnp.int32),
            pltpu.VMEM((NCH, CH), jnp.int32),
            pltpu.VMEM((CH, D), jnp.bfloat16),
            pltpu.VMEM((NP,), jnp.float32),
            pltpu.VMEM((ZBR, D), jnp.bfloat16),
            pltpu.VMEM_SHARED((NP, D), jnp.bfloat16),
        ),
        compiler_params=params,
    )
    sc_agg2 = pl.kernel(
        _sc_agg2_body,
        out_type=jax.ShapeDtypeStruct((2 * NW * N,), jnp.float32),
        mesh=mesh,
        scratch_types=(
            pltpu.VMEM((EPW,), jnp.int32),
            pltpu.VMEM((EPW,), jnp.int32),
            pltpu.VMEM((N,), jnp.float32),
            pltpu.VMEM((N,), jnp.float32),
            pltpu.VMEM((N,), jnp.float32),
            pltpu.VMEM((N,), jnp.float32),
        ),
        compiler_params=params,
    )
    return sc_agg1, sc_agg2


def kernel(x, edge_index, W1_l, b1, W1_r, W2_l, b2, W2_r):
    sc_agg1, sc_agg2 = _build_sc_kernels()
    xbf = x.astype(jnp.bfloat16)
    zrows = jnp.zeros((ZBR, D), jnp.bfloat16)
    src4 = edge_index[0].reshape(NC, NS, NCH, CH)
    dst4 = edge_index[1].reshape(NC, NS, NCH, CH)
    msum1p, cntf = sc_agg1(xbf, zrows, src4, dst4)
    cnt = cntf.reshape(NC * NS, NP)
    pt = _tc1(xbf, msum1p, cnt, W1_l, b1.reshape(1, H), W1_r, W2_l, W2_r)
    m2f = sc_agg2(pt[0], pt[1], edge_index[0], edge_index[1])
    m2 = m2f.reshape(2 * NW, N)
    outT = _tc2(m2, pt, b2.reshape(1, C))
    return outT.T


# speedup vs baseline: 1.1466x; 1.1466x over previous
import functools

import jax
import jax.numpy as jnp
from jax import lax
from jax.experimental import pallas as pl
from jax.experimental.pallas import tpu as pltpu
from jax.experimental.pallas import tpu_sc as plsc

N = 10000
E = 320000
D = 128
H = 256
C = 2

NC = 2
NS = 16
NW = NC * NS
LANES = 16

EPW = E // NW
CH = 80
NCH = EPW // CH
NP = 10240
RPT = NP // NS
ZBR = 128


def _sc_agg1_body(x_hbm, zrows_hbm, src_hbm, dst_hbm, msum_out, cnt_out,
                  src_v, dst_v, buf0, buf1, cnt_v, zrow_v, sem0, sem1, acc_s):
    cid = lax.axis_index("c")
    sid = lax.axis_index("s")

    zeros16 = jnp.zeros((LANES,), jnp.float32)
    ones16 = jnp.ones((LANES,), jnp.float32)

    pltpu.sync_copy(zrows_hbm, zrow_v)

    def _zcnt(i, c):
        cnt_v[pl.ds(i * LANES, LANES)] = zeros16
        return c
    lax.fori_loop(0, NP // LANES, _zcnt, 0)

    def _zacc(k, c):
        pltpu.sync_copy(zrow_v, acc_s.at[pl.ds(sid * RPT + k * ZBR, ZBR)])
        return c
    lax.fori_loop(0, RPT // ZBR, _zacc, 0)

    plsc.subcore_barrier()

    pltpu.sync_copy(src_hbm.at[cid].at[sid], src_v)
    pltpu.sync_copy(dst_hbm.at[cid].at[sid], dst_v)

    def _counts(j, c):
        def _cstep(k, c2):
            idx = dst_v[j, pl.ds(k * LANES, LANES)]
            plsc.addupdate_scatter(cnt_v, [idx], ones16)
            return c2
        lax.fori_loop(0, CH // LANES, _cstep, 0)
        return c

    pltpu.async_copy(x_hbm.at[src_v.at[0]], buf0, sem0)

    def _pair(t, c):
        j = 2 * t
        pltpu.make_async_copy(x_hbm.at[src_v.at[j]], buf0, sem0).wait()
        pltpu.async_copy(x_hbm.at[src_v.at[j + 1]], buf1, sem1)
        pltpu.sync_copy(buf0, acc_s.at[dst_v.at[j]], add=True)
        _counts(j, 0)
        pltpu.make_async_copy(x_hbm.at[src_v.at[j + 1]], buf1, sem1).wait()
        pltpu.async_copy(x_hbm.at[src_v.at[j + 2]], buf0, sem0)
        pltpu.sync_copy(buf1, acc_s.at[dst_v.at[j + 1]], add=True)
        _counts(j + 1, 0)
        return c
    lax.fori_loop(0, (NCH - 1) // 2, _pair, 0)

    pltpu.make_async_copy(x_hbm.at[src_v.at[NCH - 1]], buf0, sem0).wait()
    pltpu.sync_copy(buf0, acc_s.at[dst_v.at[NCH - 1]], add=True)
    _counts(NCH - 1, 0)

    plsc.subcore_barrier()

    pltpu.sync_copy(acc_s.at[pl.ds(sid * RPT, RPT)],
                    msum_out.at[cid].at[pl.ds(sid * RPT, RPT)])
    pltpu.sync_copy(cnt_v, cnt_out.at[pl.ds((cid * NS + sid) * NP, NP)])


def _sc_agg2_body(p0_hbm, p1_hbm, src_hbm, dst_hbm, out,
                  src_v, dst_v, p0_v, p1_v, a0_v, a1_v):
    cid = lax.axis_index("c")
    sid = lax.axis_index("s")
    wid = sid * NC + cid

    zeros16 = jnp.zeros((LANES,), jnp.float32)

    def _zacc(i, c):
        a0_v[pl.ds(i * LANES, LANES)] = zeros16
        a1_v[pl.ds(i * LANES, LANES)] = zeros16
        return c
    lax.fori_loop(0, N // LANES, _zacc, 0)

    pltpu.sync_copy(src_hbm.at[pl.ds(wid * EPW, EPW)], src_v)
    pltpu.sync_copy(dst_hbm.at[pl.ds(wid * EPW, EPW)], dst_v)
    pltpu.sync_copy(p0_hbm, p0_v)
    pltpu.sync_copy(p1_hbm, p1_v)

    def _step(t, c):
        s_idx = src_v[pl.ds(t * LANES, LANES)]
        d_idx = dst_v[pl.ds(t * LANES, LANES)]
        v0 = plsc.load_gather(p0_v, [s_idx])
        v1 = plsc.load_gather(p1_v, [s_idx])
        plsc.addupdate_scatter(a0_v, [d_idx], v0)
        plsc.addupdate_scatter(a1_v, [d_idx], v1)
        return c
    lax.fori_loop(0, EPW // LANES, _step, 0)

    pltpu.sync_copy(a0_v, out.at[pl.ds(wid * N, N)])
    pltpu.sync_copy(a1_v, out.at[pl.ds((NW + wid) * N, N)])


def _tc1_body(x_ref, ms_ref, cnt_ref, w1l_ref, b1_ref, w1r_ref, w2l_ref,
              w2r_ref, pt_ref):
    cnt = jnp.sum(cnt_ref[...], axis=0)[:N]
    inv = jnp.where(cnt > 0.0, 1.0 / jnp.maximum(cnt, 1.0), 0.0)
    ms = (ms_ref[0][:N].astype(jnp.float32) +
          ms_ref[1][:N].astype(jnp.float32))
    agg = ms * inv[:, None]
    h = lax.dot_general(agg, w1l_ref[...], (((1,), (1,)), ((), ())),
                        preferred_element_type=jnp.float32)
    h = h + b1_ref[...]
    h = h + lax.dot_general(x_ref[...].astype(jnp.float32), w1r_ref[...],
                            (((1,), (1,)), ((), ())),
                            preferred_element_type=jnp.float32)
    h = jnp.maximum(h, 0.0)
    p01 = lax.dot_general(w2l_ref[...], h, (((1,), (1,)), ((), ())),
                          preferred_element_type=jnp.float32)
    hr = lax.dot_general(w2r_ref[...], h, (((1,), (1,)), ((), ())),
                         preferred_element_type=jnp.float32)
    pt_ref[...] = jnp.concatenate(
        [p01, inv[None, :], hr, jnp.zeros((3, h.shape[0]), jnp.float32)],
        axis=0)


def _tc1(x, msum, cnt, W1_l, b1, W1_r, W2_l, W2_r):
    return pl.pallas_call(
        _tc1_body,
        out_shape=jax.ShapeDtypeStruct((8, N), jnp.float32),
    )(x, msum, cnt, W1_l, b1, W1_r, W2_l, W2_r)


def _tc2_body(m2_ref, pt_ref, b2_ref, out_ref):
    m2 = m2_ref[...]
    m0 = jnp.sum(m2[:NW], axis=0)
    m1 = jnp.sum(m2[NW:], axis=0)
    inv = pt_ref[2, :]
    z0 = m0 * inv + b2_ref[0, 0] + pt_ref[3, :]
    z1 = m1 * inv + b2_ref[0, 1] + pt_ref[4, :]
    mx = jnp.maximum(z0, z1)
    lse = mx + jnp.log(jnp.exp(z0 - mx) + jnp.exp(z1 - mx))
    out_ref[...] = jnp.concatenate([(z0 - lse)[None, :], (z1 - lse)[None, :]],
                                   axis=0)


def _tc2(m2, pt, b2):
    return pl.pallas_call(
        _tc2_body,
        out_shape=jax.ShapeDtypeStruct((C, N), jnp.float32),
    )(m2, pt, b2)


@functools.lru_cache(maxsize=1)
def _build_sc_kernels():
    mesh = plsc.VectorSubcoreMesh(
        core_axis_name="c", subcore_axis_name="s",
        num_cores=NC, num_subcores=NS)
    params = pltpu.CompilerParams(
        use_tc_tiling_on_sc=False, needs_layout_passes=False)
    sc_agg1 = pl.kernel(
        _sc_agg1_body,
        out_type=(
            jax.ShapeDtypeStruct((NC, NP, D), jnp.bfloat16),
            jax.ShapeDtypeStruct((NC * NS * NP,), jnp.float32),
        ),
        mesh=mesh,
        scratch_types=(
            pltpu.VMEM((NCH, CH), jnp.int32),
            pltpu.VMEM((NCH, CH), jnp.int32),
            pltpu.VMEM((CH, D), jnp.bfloat16),
            pltpu.VMEM((CH, D), jnp.bfloat16),
            pltpu.VMEM((NP,), jnp.float32),
            pltpu.VMEM((ZBR, D), jnp.bfloat16),
            pltpu.SemaphoreType.DMA,
            pltpu.SemaphoreType.DMA,
            pltpu.VMEM_SHARED((NP, D), jnp.bfloat16),
        ),
        compiler_params=params,
    )
    sc_agg2 = pl.kernel(
        _sc_agg2_body,
        out_type=jax.ShapeDtypeStruct((2 * NW * N,), jnp.float32),
        mesh=mesh,
        scratch_types=(
            pltpu.VMEM((EPW,), jnp.int32),
            pltpu.VMEM((EPW,), jnp.int32),
            pltpu.VMEM((N,), jnp.float32),
            pltpu.VMEM((N,), jnp.float32),
            pltpu.VMEM((N,), jnp.float32),
            pltpu.VMEM((N,), jnp.float32),
        ),
        compiler_params=params,
    )
    return sc_agg1, sc_agg2


def kernel(x, edge_index, W1_l, b1, W1_r, W2_l, b2, W2_r):
    sc_agg1, sc_agg2 = _build_sc_kernels()
    xbf = x.astype(jnp.bfloat16)
    zrows = jnp.zeros((ZBR, D), jnp.bfloat16)
    src4 = edge_index[0].reshape(NC, NS, NCH, CH)
    dst4 = edge_index[1].reshape(NC, NS, NCH, CH)
    msum1p, cntf = sc_agg1(xbf, zrows, src4, dst4)
    cnt = cntf.reshape(NC * NS, NP)
    pt = _tc1(xbf, msum1p, cnt, W1_l, b1.reshape(1, H), W1_r, W2_l, W2_r)
    m2f = sc_agg2(pt[0], pt[1], edge_index[0], edge_index[1])
    m2 = m2f.reshape(2 * NW, N)
    outT = _tc2(m2, pt, b2.reshape(1, C))
    return outT.T
